# baseline (speedup 1.0000x reference)
import functools

import jax
import jax.numpy as jnp
import numpy as np
from jax import lax
from jax.experimental import pallas as pl
from jax.experimental.pallas import tpu as pltpu
from jax.experimental.pallas import tpu_sc as plsc

_RADIUS = 60
_R = _RADIUS - 1
_OFFSETS = [(-1, -1), (-1, 0), (0, -1), (0, 0), (0, 1), (1, 0), (1, 1)]
_G = 2 * _R + 3
_ND = _G * _G
_SHIFTS = [_G * di + dk for (di, dk) in _OFFSETS]

_C = 8192
_NCHUNK = -(-_ND // _C)
_NDPAD = _NCHUNK * _C


def _build_rows():
    rows = []
    s = 0
    for i in range(-_R, _R + 1):
        kmin = max(-_R, i - _R)
        ln = min(_R, i + _R) - kmin + 1
        d0 = (i + _R + 1) * _G + (kmin + _R + 1)
        rows.append((s, d0, ln))
        s += ln
    return rows, s


_ROWS, _N = _build_rows()


_NSC = plsc.get_sparse_core_info()
_NW = _NSC.num_cores * _NSC.num_subcores


_CHUNK = 128
_BATCH = 8
_TPB = _NW // _BATCH
_CPB = (_N + _CHUNK - 1) // _CHUNK
_CPT = (_CPB + _TPB - 1) // _TPB


def _chunk_starts():
    starts = list(range(0, _N - _CHUNK + 1, _CHUNK))
    starts.append(_N - _CHUNK)
    while len(starts) < _TPB * _CPT:
        starts.append(_N - _CHUNK)
    return starts


_NGAP = _NDPAD - _N
_GPT = ((_NGAP + _CHUNK - 1) // _CHUNK + _TPB - 1) // _TPB


def _build_idx_tables():
    dense_idx = np.zeros(_N, np.int32)
    for (s, d0, ln) in _ROWS:
        dense_idx[s:s + ln] = d0 + np.arange(ln, dtype=np.int32)
    gaps = np.setdiff1d(np.arange(_NDPAD, dtype=np.int32), dense_idx)
    gaps = np.concatenate(
        [gaps, np.full(_TPB * _GPT * _CHUNK - len(gaps), gaps[-1], np.int32)])
    starts = _chunk_starts()
    to_dense = np.zeros((_NW, 2 * _CPT + _GPT, _CHUNK), np.int32)
    to_ragged = np.zeros((_NW, 2 * _CPT, _CHUNK), np.int32)
    for w in range(_NW):
        s = w % _TPB
        for c in range(_CPT):
            rows = starts[s * _CPT + c] + np.arange(_CHUNK, dtype=np.int32)
            to_dense[w, 2 * c] = rows
            to_dense[w, 2 * c + 1] = dense_idx[rows]
            to_ragged[w, 2 * c] = dense_idx[rows]
            to_ragged[w, 2 * c + 1] = rows
        for g in range(_GPT):
            gc = (s * _GPT + g) * _CHUNK
            to_dense[w, 2 * _CPT + g] = gaps[gc:gc + _CHUNK]
    return to_dense, to_ragged


_IDX_TO_DENSE, _IDX_TO_RAGGED = _build_idx_tables()


def _make_mover(out_rows, feat, gap_slots=0):
    mesh = plsc.VectorSubcoreMesh(core_axis_name="c", subcore_axis_name="s")
    nslot = 2 * _CPT + gap_slots
    scratch = [
        pltpu.VMEM((nslot, _CHUNK), jnp.int32),
        pltpu.VMEM((_CHUNK, feat), jnp.float32),
        pltpu.VMEM((_CHUNK, feat), jnp.float32),
        pltpu.SemaphoreType.DMA,
        pltpu.SemaphoreType.DMA,
    ]
    if gap_slots:
        scratch += [pltpu.VMEM((_CHUNK, feat), jnp.float32),
                    pltpu.SemaphoreType.DMA]

    @functools.partial(
        pl.kernel, mesh=mesh,
        out_type=jax.ShapeDtypeStruct((_BATCH, out_rows, feat), jnp.float32),
        scratch_types=scratch)
    def mover(src_hbm, idx_hbm, *rest):
        if gap_slots:
            (zero_hbm, out_hbm, idx_v, buf0, buf1, gsem, ssem,
             zbuf, zsem) = rest
        else:
            out_hbm, idx_v, buf0, buf1, gsem, ssem = rest
        wid = lax.axis_index("s") * _NSC.num_cores + lax.axis_index("c")
        pltpu.sync_copy(idx_hbm.at[wid], idx_v)
        b = wid // _TPB
        src_b = src_hbm.at[b]
        out_b = out_hbm.at[b]
        zputs = []
        if gap_slots:
            pltpu.sync_copy(zero_hbm, zbuf)
            zputs = [pltpu.make_async_copy(
                zbuf, out_b.at[idx_v.at[2 * _CPT + g]], zsem)
                for g in range(gap_slots)]
            for z in zputs:
                z.start()
        bufs = [buf0, buf1]
        gets = [pltpu.make_async_copy(src_b.at[idx_v.at[2 * c]],
                                      bufs[c % 2], gsem)
                for c in range(_CPT)]
        puts = [pltpu.make_async_copy(bufs[c % 2],
                                      out_b.at[idx_v.at[2 * c + 1]], ssem)
                for c in range(_CPT)]
        gets[0].start()
        for c in range(_CPT):
            gets[c].wait()
            puts[c].start()
            if c + 1 < _CPT:
                if c >= 1:
                    puts[c - 1].wait()
                gets[c + 1].start()
        puts[_CPT - 2].wait()
        puts[_CPT - 1].wait()
        for z in zputs:
            z.wait()

    return mover


_H = 128


def _conv_body(xc_ref, w_ref, b_ref, o_ref, ring_ref):
    j = pl.program_id(1)
    cur, prv, nxt = lax.rem(j, 3), lax.rem(j + 2, 3), lax.rem(j + 1, 3)
    xb = xc_ref[0].astype(jnp.bfloat16)
    ring_ref[cur, _H:_H + _C] = xb
    ring_ref[prv, _H + _C:2 * _H + _C] = xb[:_H]
    ring_ref[nxt, 0:_H] = xb[_C - _H:]

    @pl.when(j >= 1)
    def _compute():
        acc = jnp.broadcast_to(b_ref[0],
                               (_C, o_ref.shape[2])).astype(jnp.float32)
        for t, s in enumerate(_SHIFTS):
            xs = ring_ref[prv, _H + s:_H + s + _C]
            acc = acc + jnp.dot(xs, w_ref[t],
                                preferred_element_type=jnp.float32)
        o_ref[0] = acc


def _hexconv_dense(xd, kernel_weights, bias2d):
    batch, _, feat = xd.shape
    out_dim = kernel_weights.shape[2]
    grid = (batch, _NCHUNK + 1)
    ic = pl.BlockSpec((1, _C, feat),
                      lambda b, j: (b, jnp.minimum(j, _NCHUNK - 1), 0))
    wspec = pl.BlockSpec(kernel_weights.shape, lambda b, j: (0, 0, 0))
    bspec = pl.BlockSpec((1, out_dim), lambda b, j: (0, 0))
    ospec = pl.BlockSpec((1, _C, out_dim),
                         lambda b, j: (b, jnp.maximum(j - 1, 0), 0))
    return pl.pallas_call(
        _conv_body,
        grid=grid,
        in_specs=[ic, wspec, bspec],
        out_specs=ospec,
        out_shape=jax.ShapeDtypeStruct((batch, _NDPAD, out_dim), jnp.float32),
        scratch_shapes=[pltpu.VMEM((3, _C + 2 * _H, feat), jnp.bfloat16)],
        compiler_params=pltpu.CompilerParams(
            dimension_semantics=("parallel", "arbitrary")),
    )(xd, kernel_weights.astype(jnp.bfloat16), bias2d)


def kernel(inputs, kernel_weights, bias):
    feat = inputs.shape[2]
    out_dim = kernel_weights.shape[2]
    to_dense = _make_mover(_NDPAD, feat, gap_slots=_GPT)
    to_ragged = _make_mover(_N, out_dim)
    zeros = jnp.zeros((_CHUNK, feat), jnp.float32)
    xd = to_dense(inputs, jnp.asarray(_IDX_TO_DENSE), zeros)
    yd = _hexconv_dense(xd, kernel_weights, bias.reshape(1, -1))
    return to_ragged(yd, jnp.asarray(_IDX_TO_RAGGED))


# speedup vs baseline: 1.0837x; 1.0837x over previous
import functools

import jax
import jax.numpy as jnp
import numpy as np
from jax import lax
from jax.experimental import pallas as pl
from jax.experimental.pallas import tpu as pltpu
from jax.experimental.pallas import tpu_sc as plsc

_RADIUS = 60
_R = _RADIUS - 1
_OFFSETS = [(-1, -1), (-1, 0), (0, -1), (0, 0), (0, 1), (1, 0), (1, 1)]
_G = 2 * _R + 3
_ND = _G * _G
_SHIFTS = [_G * di + dk for (di, dk) in _OFFSETS]

_C = 4096
_NCHUNK = -(-_ND // _C)
_NDPAD = _NCHUNK * _C


def _build_rows():
    rows = []
    s = 0
    for i in range(-_R, _R + 1):
        kmin = max(-_R, i - _R)
        ln = min(_R, i + _R) - kmin + 1
        d0 = (i + _R + 1) * _G + (kmin + _R + 1)
        rows.append((s, d0, ln))
        s += ln
    return rows, s


_ROWS, _N = _build_rows()


_NSC = plsc.get_sparse_core_info()
_NW = _NSC.num_cores * _NSC.num_subcores


_CHUNK = 128
_BATCH = 8
_TPB = _NW // _BATCH
_CPB = (_N + _CHUNK - 1) // _CHUNK
_CPT = (_CPB + _TPB - 1) // _TPB


def _chunk_starts():
    starts = list(range(0, _N - _CHUNK + 1, _CHUNK))
    starts.append(_N - _CHUNK)
    while len(starts) < _TPB * _CPT:
        starts.append(_N - _CHUNK)
    return starts


_NGAP = _NDPAD - _N
_GPT = ((_NGAP + _CHUNK - 1) // _CHUNK + _TPB - 1) // _TPB


def _build_idx_tables():
    dense_idx = np.zeros(_N, np.int32)
    for (s, d0, ln) in _ROWS:
        dense_idx[s:s + ln] = d0 + np.arange(ln, dtype=np.int32)
    gaps = np.setdiff1d(np.arange(_NDPAD, dtype=np.int32), dense_idx)
    gaps = np.concatenate(
        [gaps, np.full(_TPB * _GPT * _CHUNK - len(gaps), gaps[-1], np.int32)])
    starts = _chunk_starts()
    to_dense = np.zeros((_NW, 2 * _CPT + _GPT, _CHUNK), np.int32)
    to_ragged = np.zeros((_NW, 2 * _CPT, _CHUNK), np.int32)
    for w in range(_NW):
        s = w % _TPB
        for c in range(_CPT):
            rows = starts[s * _CPT + c] + np.arange(_CHUNK, dtype=np.int32)
            to_dense[w, 2 * c] = rows
            to_dense[w, 2 * c + 1] = dense_idx[rows]
            to_ragged[w, 2 * c] = dense_idx[rows]
            to_ragged[w, 2 * c + 1] = rows
        for g in range(_GPT):
            gc = (s * _GPT + g) * _CHUNK
            to_dense[w, 2 * _CPT + g] = gaps[gc:gc + _CHUNK]
    return to_dense, to_ragged


_IDX_TO_DENSE, _IDX_TO_RAGGED = _build_idx_tables()


def _make_mover(out_rows, feat, gap_slots=0):
    mesh = plsc.VectorSubcoreMesh(core_axis_name="c", subcore_axis_name="s")
    nslot = 2 * _CPT + gap_slots
    scratch = [
        pltpu.VMEM((nslot, _CHUNK), jnp.int32),
        pltpu.VMEM((_CHUNK, feat), jnp.float32),
        pltpu.VMEM((_CHUNK, feat), jnp.float32),
        pltpu.SemaphoreType.DMA,
        pltpu.SemaphoreType.DMA,
    ]
    if gap_slots:
        scratch += [pltpu.VMEM((_CHUNK, feat), jnp.float32),
                    pltpu.SemaphoreType.DMA]

    @functools.partial(
        pl.kernel, mesh=mesh,
        out_type=jax.ShapeDtypeStruct((_BATCH, out_rows, feat), jnp.float32),
        scratch_types=scratch)
    def mover(src_hbm, idx_hbm, *rest):
        if gap_slots:
            (zero_hbm, out_hbm, idx_v, buf0, buf1, gsem, ssem,
             zbuf, zsem) = rest
        else:
            out_hbm, idx_v, buf0, buf1, gsem, ssem = rest
        wid = lax.axis_index("s") * _NSC.num_cores + lax.axis_index("c")
        pltpu.sync_copy(idx_hbm.at[wid], idx_v)
        b = wid // _TPB
        src_b = src_hbm.at[b]
        out_b = out_hbm.at[b]
        zputs = []
        if gap_slots:
            pltpu.sync_copy(zero_hbm, zbuf)
            zputs = [pltpu.make_async_copy(
                zbuf, out_b.at[idx_v.at[2 * _CPT + g]], zsem)
                for g in range(gap_slots)]
            for z in zputs:
                z.start()
        bufs = [buf0, buf1]
        gets = [pltpu.make_async_copy(src_b.at[idx_v.at[2 * c]],
                                      bufs[c % 2], gsem)
                for c in range(_CPT)]
        puts = [pltpu.make_async_copy(bufs[c % 2],
                                      out_b.at[idx_v.at[2 * c + 1]], ssem)
                for c in range(_CPT)]
        gets[0].start()
        for c in range(_CPT):
            gets[c].wait()
            puts[c].start()
            if c + 1 < _CPT:
                if c >= 1:
                    puts[c - 1].wait()
                gets[c + 1].start()
        puts[_CPT - 2].wait()
        puts[_CPT - 1].wait()
        for z in zputs:
            z.wait()

    return mover


_H = 128


def _conv_body(xc_ref, w_ref, b_ref, o_ref, ring_ref):
    j = pl.program_id(0)
    cur, prv, nxt = lax.rem(j, 3), lax.rem(j + 2, 3), lax.rem(j + 1, 3)
    xb = xc_ref[0].astype(jnp.bfloat16)
    ring_ref[cur, _H:_H + _C] = xb
    ring_ref[prv, _H + _C:2 * _H + _C] = xb[:_H]
    ring_ref[nxt, 0:_H] = xb[_C - _H:]

    @pl.when(j >= 1)
    def _compute():
        acc = jnp.broadcast_to(b_ref[0],
                               (_C, o_ref.shape[2])).astype(jnp.float32)
        for t, s in enumerate(_SHIFTS):
            xs = ring_ref[prv, _H + s:_H + s + _C]
            acc = acc + jnp.dot(xs, w_ref[t],
                                preferred_element_type=jnp.float32)
        o_ref[0] = acc


def _hexconv_dense(xd, kernel_weights, bias2d):
    batch, _, feat = xd.shape
    out_dim = kernel_weights.shape[2]
    nstream = batch * _NCHUNK
    grid = (nstream + 1,)

    def _in_map(g):
        c = jnp.minimum(g, nstream - 1)
        return (c // _NCHUNK, c % _NCHUNK, 0)

    def _out_map(g):
        m = jnp.maximum(g - 1, 0)
        return (m // _NCHUNK, m % _NCHUNK, 0)

    ic = pl.BlockSpec((1, _C, feat), _in_map)
    wspec = pl.BlockSpec(kernel_weights.shape, lambda g: (0, 0, 0))
    bspec = pl.BlockSpec((1, out_dim), lambda g: (0, 0))
    ospec = pl.BlockSpec((1, _C, out_dim), _out_map)
    return pl.pallas_call(
        _conv_body,
        grid=grid,
        in_specs=[ic, wspec, bspec],
        out_specs=ospec,
        out_shape=jax.ShapeDtypeStruct((batch, _NDPAD, out_dim), jnp.float32),
        scratch_shapes=[pltpu.VMEM((3, _C + 2 * _H, feat), jnp.bfloat16)],
        compiler_params=pltpu.CompilerParams(
            dimension_semantics=("arbitrary",)),
    )(xd, kernel_weights.astype(jnp.bfloat16), bias2d)


def kernel(inputs, kernel_weights, bias):
    feat = inputs.shape[2]
    out_dim = kernel_weights.shape[2]
    to_dense = _make_mover(_NDPAD, feat, gap_slots=_GPT)
    to_ragged = _make_mover(_N, out_dim)
    zeros = jnp.zeros((_CHUNK, feat), jnp.float32)
    xd = to_dense(inputs, jnp.asarray(_IDX_TO_DENSE), zeros)
    yd = _hexconv_dense(xd, kernel_weights, bias.reshape(1, -1))
    return to_ragged(yd, jnp.asarray(_IDX_TO_RAGGED))
